# baseline (speedup 1.0000x reference)
import functools

import jax
import jax.numpy as jnp
from jax import lax
from jax.experimental import pallas as pl
from jax.experimental.pallas import tpu as pltpu
from jax.experimental.pallas import tpu_sc as plsc

_VOCAB = 1000000
_D = 64
_BATCH = 16384
_HIST = 50
_TOTAL = _BATCH * _HIST

_NC = 2
_NS = 16
_NW = _NC * _NS
_TILE = 128
_N_TILES = _TOTAL // _TILE
_T_PER_W = _N_TILES // _NW
_B_PER_W = _T_PER_W * _TILE

_mesh = plsc.VectorSubcoreMesh(core_axis_name="c", subcore_axis_name="s")


@functools.partial(
    pl.kernel,
    mesh=_mesh,
    out_type=jax.ShapeDtypeStruct((_HIST, 8, _BATCH // 128, 8, 128),
                                  jnp.float32),
    scratch_types=[
        pltpu.VMEM((_B_PER_W,), jnp.int32),
        [pltpu.VMEM((_TILE,), jnp.int32)] * 2,
        [pltpu.VMEM((_TILE,), jnp.int32)] * 2,
        [pltpu.VMEM((_TILE, 128), jnp.float32)] * 2,
        [pltpu.VMEM((8, 8, 128), jnp.float32)] * 2,
        [pltpu.SemaphoreType.DMA] * 2,
        [pltpu.SemaphoreType.DMA] * 2,
    ],
    compiler_params=pltpu.CompilerParams(
        use_tc_tiling_on_sc=False, needs_layout_passes=False),
)
def _gk(xt_hbm, table_hbm, out_hbm, idx_all, idx_v, colb, gbuf, dbuf,
        sg, so):
    wid = lax.axis_index("s") * _NC + lax.axis_index("c")
    t0 = wid * _T_PER_W

    pltpu.sync_copy(xt_hbm.at[pl.ds(t0 * _TILE, _B_PER_W)], idx_all)

    def compute_idx(tl, b):
        for j in range(8):
            xv = idx_all[pl.ds(tl * _TILE + 16 * j, 16)]
            idx_v[b][pl.ds(16 * j, 16)] = xv >> 1
            colb[b][pl.ds(16 * j, 16)] = (xv & 1) * 64

    def start_gather(b):
        pltpu.async_copy(table_hbm.at[idx_v[b]], gbuf[b], sg[b])

    def wait_gather(b):
        pltpu.make_async_copy(table_hbm.at[idx_v[b]], gbuf[b], sg[b]).wait()

    def out_slice(tg, dblk):
        h = tg // 128
        bb = lax.rem(tg, 128)
        return out_hbm.at[h, dblk, bb]

    def start_store(tg, b):
        for dblk in range(8):
            pltpu.async_copy(dbuf[b].at[dblk], out_slice(tg, dblk), so[b])

    def wait_store(tg, b):
        for dblk in range(8):
            pltpu.make_async_copy(dbuf[b].at[dblk], out_slice(tg, dblk),
                                  so[b]).wait()

    def transpose_tile(b):
        def dblk_body(dblk, carry):
            for j in range(8):
                rows = lax.iota(jnp.int32, 16) + (16 * j)
                cb = colb[b][pl.ds(16 * j, 16)] + dblk * 8
                for dl in range(8):
                    val = plsc.load_gather(gbuf[b], [rows, cb + dl])
                    dbuf[b][dblk, dl, pl.ds(16 * j, 16)] = val
            return carry

        lax.fori_loop(0, 8, dblk_body, 0)

    compute_idx(0, 0)
    start_gather(0)

    def body(u, carry):
        for b in range(2):
            t = 2 * u + b
            nb = 1 - b

            @pl.when(t + 1 < _T_PER_W)
            def _():
                compute_idx(t + 1, nb)
                start_gather(nb)

            wait_gather(b)

            @pl.when(t >= 2)
            def _():
                wait_store(t0 + t - 2, b)

            transpose_tile(b)
            start_store(t0 + t, b)
        return carry

    lax.fori_loop(0, _T_PER_W // 2, body, 0)

    wait_store(t0 + _T_PER_W - 2, 0)
    wait_store(t0 + _T_PER_W - 1, 1)


def kernel(x, weight):
    w2 = weight.reshape(_VOCAB // 2, 128)
    xtf = x.T.reshape(_TOTAL)
    o5 = _gk(xtf, w2)
    return o5.transpose(2, 4, 0, 1, 3).reshape(_BATCH, _HIST, _D)


# speedup vs baseline: 1.0528x; 1.0528x over previous
import functools

import jax
import jax.numpy as jnp
from jax import lax
from jax.experimental import pallas as pl
from jax.experimental.pallas import tpu as pltpu
from jax.experimental.pallas import tpu_sc as plsc

_VOCAB = 1000000
_D = 64
_BATCH = 16384
_HIST = 50
_TOTAL = _BATCH * _HIST

_NC = 2
_NS = 16
_NW = _NC * _NS
_TILE = 128
_N_TILES = _TOTAL // _TILE
_T_PER_W = _N_TILES // _NW
_B_PER_W = _T_PER_W * _TILE

_mesh = plsc.VectorSubcoreMesh(core_axis_name="c", subcore_axis_name="s")


@functools.partial(
    pl.kernel,
    mesh=_mesh,
    out_type=jax.ShapeDtypeStruct((_HIST, 8, _BATCH // 128, 8, 128),
                                  jnp.float32),
    scratch_types=[
        pltpu.VMEM((_B_PER_W,), jnp.int32),
        [pltpu.VMEM((_TILE, _D), jnp.float32)] * 2,
        [pltpu.VMEM((8, 8, 128), jnp.float32)] * 2,
        [pltpu.SemaphoreType.DMA] * 2,
        [pltpu.SemaphoreType.DMA] * 2,
    ],
    compiler_params=pltpu.CompilerParams(
        use_tc_tiling_on_sc=False, needs_layout_passes=False),
)
def _gk(xt_hbm, table_hbm, out_hbm, idx_all, gbuf, dbuf, sg, so):
    wid = lax.axis_index("s") * _NC + lax.axis_index("c")
    t0 = wid * _T_PER_W

    pltpu.sync_copy(xt_hbm.at[pl.ds(t0 * _TILE, _B_PER_W)], idx_all)

    def idx_ref(tl):
        return idx_all.at[pl.ds(tl * _TILE, _TILE)]

    def start_gather(tl, b):
        pltpu.async_copy(table_hbm.at[idx_ref(tl)], gbuf[b], sg[b])

    def wait_gather(tl, b):
        pltpu.make_async_copy(table_hbm.at[idx_ref(tl)], gbuf[b],
                              sg[b]).wait()

    def out_slice(tg, dblk):
        h = tg // 128
        bb = lax.rem(tg, 128)
        return out_hbm.at[h, dblk, bb]

    def start_store(tg, b):
        for dblk in range(8):
            pltpu.async_copy(dbuf[b].at[dblk], out_slice(tg, dblk), so[b])

    def wait_store(tg, b):
        for dblk in range(8):
            pltpu.make_async_copy(dbuf[b].at[dblk], out_slice(tg, dblk),
                                  so[b]).wait()

    riota = [lax.iota(jnp.int32, 16) + 16 * j for j in range(8)]

    def transpose_tile(b):
        def dblk_body(dblk, carry):
            c0 = dblk * 8
            for dl in range(8):
                col = jnp.full((16,), c0 + dl, jnp.int32)
                for j in range(8):
                    val = plsc.load_gather(gbuf[b], [riota[j], col])
                    dbuf[b][dblk, dl, pl.ds(16 * j, 16)] = val
            return carry

        lax.fori_loop(0, 8, dblk_body, 0)

    start_gather(0, 0)

    def body(u, carry):
        for b in range(2):
            t = 2 * u + b
            nb = 1 - b

            @pl.when(t + 1 < _T_PER_W)
            def _():
                start_gather(t + 1, nb)

            wait_gather(t, b)

            @pl.when(t >= 2)
            def _():
                wait_store(t0 + t - 2, b)

            transpose_tile(b)
            start_store(t0 + t, b)
        return carry

    lax.fori_loop(0, _T_PER_W // 2, body, 0)

    wait_store(t0 + _T_PER_W - 2, 0)
    wait_store(t0 + _T_PER_W - 1, 1)


def kernel(x, weight):
    xtf = x.T.reshape(_TOTAL)
    o5 = _gk(xtf, weight)
    return o5.transpose(2, 4, 0, 1, 3).reshape(_BATCH, _HIST, _D)


# speedup vs baseline: 1.5503x; 1.4726x over previous
import functools

import jax
import jax.numpy as jnp
from jax import lax
from jax.experimental import pallas as pl
from jax.experimental.pallas import tpu as pltpu
from jax.experimental.pallas import tpu_sc as plsc

_VOCAB = 1000000
_N_HIDDEN = 64
_BATCH = 16384
_HIST = 50
_TOTAL = _BATCH * _HIST

_NC = 2
_NS = 16
_NW = _NC * _NS
_B_PER_W = _TOTAL // _NW
_CHUNK = 400
_N_CHUNKS = _B_PER_W // _CHUNK
_NBUF = 4
_N_GROUPS = _N_CHUNKS // _NBUF

_mesh = plsc.VectorSubcoreMesh(core_axis_name="c", subcore_axis_name="s")


@functools.partial(
    pl.kernel,
    mesh=_mesh,
    out_type=jax.ShapeDtypeStruct((_TOTAL, _N_HIDDEN), jnp.float32),
    scratch_types=[
        pltpu.VMEM((_B_PER_W,), jnp.int32),
        [pltpu.VMEM((_CHUNK, _N_HIDDEN), jnp.float32)] * _NBUF,
        [pltpu.SemaphoreType.DMA] * _NBUF,
        [pltpu.SemaphoreType.DMA] * _NBUF,
    ],
    compiler_params=pltpu.CompilerParams(use_tc_tiling_on_sc=False),
)
def _gather_kernel(idx_hbm, table_hbm, out_hbm, idx_all, bufs, sgs, sos):
    wid = lax.axis_index("s") * _NC + lax.axis_index("c")
    base = wid * _B_PER_W

    pltpu.sync_copy(idx_hbm.at[pl.ds(base, _B_PER_W)], idx_all)

    def start_gather(i, b):
        pltpu.async_copy(
            table_hbm.at[idx_all.at[pl.ds(i * _CHUNK, _CHUNK)]], bufs[b],
            sgs[b])

    def wait_gather(i, b):
        pltpu.make_async_copy(
            table_hbm.at[idx_all.at[pl.ds(i * _CHUNK, _CHUNK)]], bufs[b],
            sgs[b]).wait()

    def start_store(i, b):
        pltpu.async_copy(bufs[b], out_hbm.at[pl.ds(base + i * _CHUNK, _CHUNK)],
                         sos[b])

    def wait_store(i, b):
        pltpu.make_async_copy(
            bufs[b], out_hbm.at[pl.ds(base + i * _CHUNK, _CHUNK)],
            sos[b]).wait()

    for b in range(_NBUF):
        start_gather(b, b)

    def body(j, carry):
        for b in range(_NBUF):
            i = _NBUF * j + b
            wait_gather(i, b)
            start_store(i, b)
            ip = i - 1
            pb = (b - 1) % _NBUF

            @pl.when((ip >= 0) & (ip < _N_CHUNKS - _NBUF))
            def _():
                wait_store(ip, pb)
                start_gather(ip + _NBUF, pb)

        return carry

    lax.fori_loop(0, _N_GROUPS, body, 0)

    for b in range(_NBUF):
        wait_store(_N_CHUNKS - _NBUF + b, b)


def kernel(x, weight):
    flat = x.reshape(_TOTAL).astype(jnp.int32)
    out = _gather_kernel(flat, weight)
    return out.reshape(_BATCH, _HIST, _N_HIDDEN)


# speedup vs baseline: 1.9963x; 1.2877x over previous
import functools

import jax
import jax.numpy as jnp
from jax import lax
from jax.experimental import pallas as pl
from jax.experimental.pallas import tpu as pltpu
from jax.experimental.pallas import tpu_sc as plsc

_VOCAB = 1000000
_D = 64
_BATCH = 16384
_HIST = 50
_TOTAL = _BATCH * _HIST

_NC = 2
_NS = 16
_NW = _NC * _NS
_B_PER_W = _TOTAL // _NW
_CHUNK = 256
_N_CHUNKS = _B_PER_W // _CHUNK
_NBUF = 4
_N_GROUPS = _N_CHUNKS // _NBUF
_GROUPS = _TOTAL // 256

_mesh = plsc.VectorSubcoreMesh(core_axis_name="c", subcore_axis_name="s")


@functools.partial(
    pl.kernel,
    mesh=_mesh,
    out_type=jax.ShapeDtypeStruct((_GROUPS, 128, 128), jnp.float32),
    scratch_types=[
        pltpu.VMEM((_B_PER_W,), jnp.int32),
        [pltpu.VMEM((_CHUNK, _D), jnp.float32)] * _NBUF,
        [pltpu.SemaphoreType.DMA] * _NBUF,
        [pltpu.SemaphoreType.DMA] * _NBUF,
    ],
    compiler_params=pltpu.CompilerParams(use_tc_tiling_on_sc=False),
)
def _gather_kernel(idx_hbm, table_hbm, out_hbm, idx_all, bufs, sgs, sos):
    wid = lax.axis_index("s") * _NC + lax.axis_index("c")
    base = wid * _B_PER_W
    g0 = wid * _N_CHUNKS

    pltpu.sync_copy(idx_hbm.at[pl.ds(base, _B_PER_W)], idx_all)

    def start_gather(i, b):
        pltpu.async_copy(
            table_hbm.at[idx_all.at[pl.ds(i * _CHUNK, _CHUNK)]], bufs[b],
            sgs[b])

    def wait_gather(i, b):
        pltpu.make_async_copy(
            table_hbm.at[idx_all.at[pl.ds(i * _CHUNK, _CHUNK)]], bufs[b],
            sgs[b]).wait()

    def start_store(i, b):
        for qq in range(2):
            pltpu.async_copy(
                bufs[b].at[pl.ds(128 * qq, 128)],
                out_hbm.at[g0 + i, :, pl.ds(_D * qq, _D)], sos[b])

    def wait_store(i, b):
        for qq in range(2):
            pltpu.make_async_copy(
                bufs[b].at[pl.ds(128 * qq, 128)],
                out_hbm.at[g0 + i, :, pl.ds(_D * qq, _D)], sos[b]).wait()

    for b in range(_NBUF):
        start_gather(b, b)

    def body(j, carry):
        for b in range(_NBUF):
            i = _NBUF * j + b
            wait_gather(i, b)
            start_store(i, b)
            ip = i - 1
            pb = (b - 1) % _NBUF

            @pl.when((ip >= 0) & (ip < _N_CHUNKS - _NBUF))
            def _():
                wait_store(ip, pb)
                start_gather(ip + _NBUF, pb)

        return carry

    lax.fori_loop(0, _N_GROUPS, body, 0)

    for b in range(_NBUF):
        wait_store(_N_CHUNKS - _NBUF + b, b)


def _tc_body(x_ref, o_ref):
    for mm in range(16):
        xx = x_ref[pl.ds(128 * mm, 128), :]
        t2 = jnp.transpose(xx)
        for qq in range(2):
            o_ref[0, :, 2 * mm + qq, :, :] = (
                t2[64 * qq:64 * qq + 64, :].reshape(8, 8, 128))


_tc_transpose = pl.pallas_call(
    _tc_body,
    grid=(_GROUPS // 16,),
    in_specs=[pl.BlockSpec((2048, 128), lambda t: (t, 0))],
    out_specs=pl.BlockSpec((1, 8, 32, 8, 128),
                           lambda t: (t // 4, 0, t % 4, 0, 0)),
    out_shape=jax.ShapeDtypeStruct((_HIST, 8, _BATCH // 128, 8, 128),
                                   jnp.float32),
)


def kernel(x, weight):
    xtf = x.T.reshape(_TOTAL)
    g = _gather_kernel(xtf, weight)
    o5 = _tc_transpose(g.reshape(_GROUPS * 128, 128))
    return o5.transpose(2, 4, 0, 1, 3).reshape(_BATCH, _HIST, _D)


# speedup vs baseline: 2.1277x; 1.0658x over previous
import functools

import jax
import jax.numpy as jnp
from jax import lax
from jax.experimental import pallas as pl
from jax.experimental.pallas import tpu as pltpu
from jax.experimental.pallas import tpu_sc as plsc

_VOCAB = 1000000
_D = 64
_BATCH = 16384
_HIST = 50
_TOTAL = _BATCH * _HIST

_NC = 2
_NS = 16
_NW = _NC * _NS
_B_PER_W = _TOTAL // _NW
_CHUNK = 256
_N_CHUNKS = _B_PER_W // _CHUNK
_NBUF = 4
_N_GROUPS = _N_CHUNKS // _NBUF
_GROUPS = _TOTAL // 256

_mesh = plsc.VectorSubcoreMesh(core_axis_name="c", subcore_axis_name="s")


@functools.partial(
    pl.kernel,
    mesh=_mesh,
    out_type=jax.ShapeDtypeStruct((_GROUPS, 128, 128), jnp.float32),
    scratch_types=[
        pltpu.VMEM((_B_PER_W,), jnp.int32),
        [pltpu.VMEM((_CHUNK, _D), jnp.float32)] * _NBUF,
        [pltpu.SemaphoreType.DMA] * _NBUF,
        [pltpu.SemaphoreType.DMA] * _NBUF,
    ],
    compiler_params=pltpu.CompilerParams(use_tc_tiling_on_sc=False),
)
def _gather_kernel(idx_hbm, table_hbm, out_hbm, idx_all, bufs, sgs, sos):
    wid = lax.axis_index("s") * _NC + lax.axis_index("c")
    base = wid * _B_PER_W
    g0 = wid * _N_CHUNKS

    pltpu.sync_copy(idx_hbm.at[pl.ds(base, _B_PER_W)], idx_all)

    def start_gather(i, b):
        pltpu.async_copy(
            table_hbm.at[idx_all.at[pl.ds(i * _CHUNK, _CHUNK)]], bufs[b],
            sgs[b])

    def wait_gather(i, b):
        pltpu.make_async_copy(
            table_hbm.at[idx_all.at[pl.ds(i * _CHUNK, _CHUNK)]], bufs[b],
            sgs[b]).wait()

    def start_store(i, b):
        for qq in range(2):
            pltpu.async_copy(
                bufs[b].at[pl.ds(128 * qq, 128)],
                out_hbm.at[g0 + i, :, pl.ds(_D * qq, _D)], sos[b])

    def wait_store(i, b):
        for qq in range(2):
            pltpu.make_async_copy(
                bufs[b].at[pl.ds(128 * qq, 128)],
                out_hbm.at[g0 + i, :, pl.ds(_D * qq, _D)], sos[b]).wait()

    for b in range(_NBUF):
        start_gather(b, b)

    def body(j, carry):
        for b in range(_NBUF):
            i = _NBUF * j + b
            wait_gather(i, b)
            start_store(i, b)
            ip = i - 1
            pb = (b - 1) % _NBUF

            @pl.when((ip >= 0) & (ip < _N_CHUNKS - _NBUF))
            def _():
                wait_store(ip, pb)
                start_gather(ip + _NBUF, pb)

        return carry

    lax.fori_loop(0, _N_GROUPS, body, 0)

    for b in range(_NBUF):
        wait_store(_N_CHUNKS - _NBUF + b, b)


def _tc_body(x_ref, o_ref):
    for mm in range(32):
        xx = x_ref[pl.ds(128 * mm, 128), :]
        t2 = jnp.transpose(xx)
        o_ref[0, :, 2 * mm:2 * mm + 2, :, :] = jnp.transpose(
            t2.reshape(2, 8, 8, 128), (1, 0, 2, 3))


_tc_transpose = pl.pallas_call(
    _tc_body,
    grid=(_GROUPS // 32,),
    in_specs=[pl.BlockSpec((4096, 128), lambda t: (t, 0))],
    out_specs=pl.BlockSpec((1, 8, 64, 8, 128),
                           lambda t: (t // 2, 0, t % 2, 0, 0)),
    out_shape=jax.ShapeDtypeStruct((_HIST, 8, _BATCH // 128, 8, 128),
                                   jnp.float32),
)


def kernel(x, weight):
    xtf = x.T.reshape(_TOTAL)
    g = _gather_kernel(xtf, weight)
    o5 = _tc_transpose(g.reshape(_GROUPS * 128, 128))
    return o5.transpose(2, 4, 0, 1, 3).reshape(_BATCH, _HIST, _D)


# speedup vs baseline: 2.1918x; 1.0301x over previous
import functools

import jax
import jax.numpy as jnp
from jax import lax
from jax.experimental import pallas as pl
from jax.experimental.pallas import tpu as pltpu
from jax.experimental.pallas import tpu_sc as plsc

_VOCAB = 1000000
_D = 64
_BATCH = 16384
_HIST = 50
_TOTAL = _BATCH * _HIST

_NC = 2
_NS = 16
_NW = _NC * _NS
_B_PER_W = _TOTAL // _NW
_CHUNK = 256
_N_CHUNKS = _B_PER_W // _CHUNK
_NBUF = 4
_N_GROUPS = _N_CHUNKS // _NBUF
_GROUPS = _TOTAL // 256

_mesh = plsc.VectorSubcoreMesh(core_axis_name="c", subcore_axis_name="s")


@functools.partial(
    pl.kernel,
    mesh=_mesh,
    out_type=jax.ShapeDtypeStruct((_GROUPS, 128, 128), jnp.float32),
    scratch_types=[
        pltpu.VMEM((_B_PER_W,), jnp.int32),
        [pltpu.VMEM((_CHUNK, _D), jnp.float32)] * _NBUF,
        [pltpu.SemaphoreType.DMA] * _NBUF,
        [pltpu.SemaphoreType.DMA] * _NBUF,
    ],
    compiler_params=pltpu.CompilerParams(use_tc_tiling_on_sc=False),
)
def _gather_kernel(idx_hbm, table_hbm, out_hbm, idx_all, bufs, sgs, sos):
    wid = lax.axis_index("s") * _NC + lax.axis_index("c")
    base = wid * _B_PER_W
    g0 = wid * _N_CHUNKS

    pltpu.sync_copy(idx_hbm.at[pl.ds(base, _B_PER_W)], idx_all)

    def start_gather(i, b):
        pltpu.async_copy(
            table_hbm.at[idx_all.at[pl.ds(i * _CHUNK, _CHUNK)]], bufs[b],
            sgs[b])

    def wait_gather(i, b):
        pltpu.make_async_copy(
            table_hbm.at[idx_all.at[pl.ds(i * _CHUNK, _CHUNK)]], bufs[b],
            sgs[b]).wait()

    def start_store(i, b):
        for qq in range(2):
            pltpu.async_copy(
                bufs[b].at[pl.ds(128 * qq, 128)],
                out_hbm.at[g0 + i, :, pl.ds(_D * qq, _D)], sos[b])

    def wait_store(i, b):
        for qq in range(2):
            pltpu.make_async_copy(
                bufs[b].at[pl.ds(128 * qq, 128)],
                out_hbm.at[g0 + i, :, pl.ds(_D * qq, _D)], sos[b]).wait()

    for b in range(_NBUF):
        start_gather(b, b)

    def body(j, carry):
        for b in range(_NBUF):
            i = _NBUF * j + b
            wait_gather(i, b)
            start_store(i, b)
            ip = i - 1
            pb = (b - 1) % _NBUF

            @pl.when((ip >= 0) & (ip < _N_CHUNKS - _NBUF))
            def _():
                wait_store(ip, pb)
                start_gather(ip + _NBUF, pb)

        return carry

    lax.fori_loop(0, _N_GROUPS, body, 0)

    for b in range(_NBUF):
        wait_store(_N_CHUNKS - _NBUF + b, b)


def _tc_body(x_ref, o_ref):
    for mm in range(64):
        xx = x_ref[pl.ds(128 * mm, 128), :]
        t2 = jnp.transpose(xx)
        o_ref[0, :, 2 * mm:2 * mm + 2, :, :] = jnp.transpose(
            t2.reshape(2, 8, 8, 128), (1, 0, 2, 3))


_tc_transpose = pl.pallas_call(
    _tc_body,
    grid=(_GROUPS // 64,),
    in_specs=[pl.BlockSpec((8192, 128), lambda t: (t, 0))],
    out_specs=pl.BlockSpec((1, 8, 128, 8, 128),
                           lambda t: (t, 0, 0, 0, 0)),
    out_shape=jax.ShapeDtypeStruct((_HIST, 8, _BATCH // 128, 8, 128),
                                   jnp.float32),
)


def kernel(x, weight):
    xtf = x.T.reshape(_TOTAL)
    g = _gather_kernel(xtf, weight)
    o5 = _tc_transpose(g.reshape(_GROUPS * 128, 128))
    return o5.transpose(2, 4, 0, 1, 3).reshape(_BATCH, _HIST, _D)


# speedup vs baseline: 2.1981x; 1.0029x over previous
import functools

import jax
import jax.numpy as jnp
from jax import lax
from jax.experimental import pallas as pl
from jax.experimental.pallas import tpu as pltpu
from jax.experimental.pallas import tpu_sc as plsc

_VOCAB = 1000000
_D = 64
_BATCH = 16384
_HIST = 50
_TOTAL = _BATCH * _HIST

_NC = 2
_NS = 16
_NW = _NC * _NS
_B_PER_W = _TOTAL // _NW
_CHUNK = 256
_N_CHUNKS = _B_PER_W // _CHUNK
_NBUF = 4
_N_GROUPS = _N_CHUNKS // _NBUF
_GROUPS = _TOTAL // 256

_mesh = plsc.VectorSubcoreMesh(core_axis_name="c", subcore_axis_name="s")


@functools.partial(
    pl.kernel,
    mesh=_mesh,
    out_type=jax.ShapeDtypeStruct((_GROUPS, 128, 128), jnp.float32),
    scratch_types=[
        pltpu.VMEM((_B_PER_W,), jnp.int32),
        [pltpu.VMEM((_CHUNK, _D), jnp.float32)] * _NBUF,
        [pltpu.SemaphoreType.DMA] * _NBUF,
        [pltpu.SemaphoreType.DMA] * _NBUF,
    ],
    compiler_params=pltpu.CompilerParams(use_tc_tiling_on_sc=False),
)
def _gather_kernel(idx_hbm, table_hbm, out_hbm, idx_all, bufs, sgs, sos):
    wid = lax.axis_index("s") * _NC + lax.axis_index("c")
    base = wid * _B_PER_W
    g0 = wid * _N_CHUNKS

    pltpu.sync_copy(idx_hbm.at[pl.ds(base, _B_PER_W)], idx_all)

    def start_gather(i, b):
        pltpu.async_copy(
            table_hbm.at[idx_all.at[pl.ds(i * _CHUNK, _CHUNK)]], bufs[b],
            sgs[b])

    def wait_gather(i, b):
        pltpu.make_async_copy(
            table_hbm.at[idx_all.at[pl.ds(i * _CHUNK, _CHUNK)]], bufs[b],
            sgs[b]).wait()

    def start_store(i, b):
        for qq in range(2):
            pltpu.async_copy(
                bufs[b].at[pl.ds(128 * qq, 128)],
                out_hbm.at[g0 + i, :, pl.ds(_D * qq, _D)], sos[b])

    def wait_store(i, b):
        for qq in range(2):
            pltpu.make_async_copy(
                bufs[b].at[pl.ds(128 * qq, 128)],
                out_hbm.at[g0 + i, :, pl.ds(_D * qq, _D)], sos[b]).wait()

    for b in range(_NBUF):
        start_gather(b, b)

    def body(j, carry):
        for b in range(_NBUF):
            i = _NBUF * j + b
            wait_gather(i, b)
            start_store(i, b)
            ip = i - 1
            pb = (b - 1) % _NBUF

            @pl.when((ip >= 0) & (ip < _N_CHUNKS - _NBUF))
            def _():
                wait_store(ip, pb)
                start_gather(ip + _NBUF, pb)

        return carry

    lax.fori_loop(0, _N_GROUPS, body, 0)

    for b in range(_NBUF):
        wait_store(_N_CHUNKS - _NBUF + b, b)


def _tc_body(x_ref, o_ref):
    for mm in range(128):
        xx = x_ref[pl.ds(128 * mm, 128), :]
        t2 = jnp.transpose(xx)
        o_ref[mm // 64, :, 2 * (mm % 64):2 * (mm % 64) + 2, :, :] = jnp.transpose(
            t2.reshape(2, 8, 8, 128), (1, 0, 2, 3))


_tc_transpose = pl.pallas_call(
    _tc_body,
    grid=(_GROUPS // 128,),
    in_specs=[pl.BlockSpec((16384, 128), lambda t: (t, 0))],
    out_specs=pl.BlockSpec((2, 8, 128, 8, 128),
                           lambda t: (t, 0, 0, 0, 0)),
    out_shape=jax.ShapeDtypeStruct((_HIST, 8, _BATCH // 128, 8, 128),
                                   jnp.float32),
)


def kernel(x, weight):
    xtf = x.T.reshape(_TOTAL)
    g = _gather_kernel(xtf, weight)
    o5 = _tc_transpose(g.reshape(_GROUPS * 128, 128))
    return o5.transpose(2, 4, 0, 1, 3).reshape(_BATCH, _HIST, _D)
